# baseline (speedup 1.0000x reference)
import functools

import jax
import jax.numpy as jnp
from jax import lax
from jax.experimental import pallas as pl
from jax.experimental.pallas import tpu as pltpu
from jax.experimental.pallas import tpu_sc as plsc

E = 8
K = 2
C = 2048
NTOK = 8192
NA = NTOK * K
TILE = 256
P = NA + E * TILE
NT = P // TILE
NC, NS = 2, 16
NW = NC * NS

GATE_TM = 1024

_DISP_B = P // NW
_DISP_G = 48
_CMB_B = NTOK // NW
_CMB_G = 16


def _gate_body(x_ref, gw_ref, sel_ref, w_ref):
    x = x_ref[...]
    logits = lax.dot_general(x, gw_ref[...], (((1,), (1,)), ((), ())),
                             preferred_element_type=jnp.float32)
    eiota = lax.broadcasted_iota(jnp.int32, logits.shape, 1)
    m1 = jnp.max(logits, axis=1, keepdims=True)
    a1 = jnp.min(jnp.where(logits == m1, eiota, E), axis=1, keepdims=True)
    l2 = jnp.where(eiota == a1, -jnp.inf, logits)
    m2 = jnp.max(l2, axis=1, keepdims=True)
    a2 = jnp.min(jnp.where(l2 == m2, eiota, E), axis=1, keepdims=True)
    z = jnp.exp(m2 - m1)
    sel_ref[...] = jnp.concatenate([a1, a2], axis=1)
    w_ref[...] = jnp.concatenate([1.0 / (1.0 + z), z / (1.0 + z)], axis=1)


def _gate(x2, gate_w):
    return pl.pallas_call(
        _gate_body,
        grid=(NTOK // GATE_TM,),
        in_specs=[pl.BlockSpec((GATE_TM, C), lambda i: (i, 0)),
                  pl.BlockSpec((E, C), lambda i: (0, 0))],
        out_specs=[pl.BlockSpec((GATE_TM, K), lambda i: (i, 0)),
                   pl.BlockSpec((GATE_TM, K), lambda i: (i, 0))],
        out_shape=[jax.ShapeDtypeStruct((NTOK, K), jnp.int32),
                   jax.ShapeDtypeStruct((NTOK, K), jnp.float32)],
    )(x2, gate_w)


def _route(sel, w):
    flat_e = sel.reshape(-1)
    onehot = (flat_e[:, None] == jnp.arange(E, dtype=jnp.int32)[None, :])
    cum = jnp.cumsum(onehot.astype(jnp.int32), axis=0)
    counts = cum[-1]
    rank = jnp.take_along_axis(cum, flat_e[:, None], axis=1)[:, 0] - 1
    aligned = ((counts + TILE - 1) // TILE) * TILE
    ends = jnp.cumsum(aligned)
    off = ends - aligned
    pos = off[flat_e] + rank
    tok = jnp.zeros((P,), jnp.int32).at[pos].set(
        jnp.arange(NA, dtype=jnp.int32) // K)
    wsort = jnp.zeros((P,), jnp.float32).at[pos].set(w.reshape(-1))
    tile_starts = jnp.arange(NT, dtype=jnp.int32) * TILE
    tile_e = jnp.minimum(
        jnp.searchsorted(ends, tile_starts, side="right"), E - 1
    ).astype(jnp.int32)
    pos2 = pos.reshape(NTOK, K).astype(jnp.int32)
    return tok, wsort.reshape(P, 1), tile_e, pos2[:, 0], pos2[:, 1]


def _dispatch_body(x_hbm, tok_hbm, xs_hbm, idx_v, rows_v, sem):
    wid = lax.axis_index("s") * NC + lax.axis_index("c")
    base = wid * _DISP_B
    pltpu.sync_copy(tok_hbm.at[pl.ds(base, _DISP_B)], idx_v)
    for g in range(_DISP_B // _DISP_G):
        pltpu.async_copy(
            x_hbm.at[idx_v.at[pl.ds(g * _DISP_G, _DISP_G)]], rows_v, sem
        ).wait()
        pltpu.sync_copy(rows_v, xs_hbm.at[pl.ds(base + g * _DISP_G, _DISP_G)])


def _dispatch(x2, tok):
    f = pl.kernel(
        _dispatch_body,
        out_type=jax.ShapeDtypeStruct((P, C), jnp.float32),
        mesh=plsc.VectorSubcoreMesh(core_axis_name="c", subcore_axis_name="s",
                                    num_cores=NC, num_subcores=NS),
        scratch_types=[pltpu.VMEM((_DISP_B,), jnp.int32),
                       pltpu.VMEM((_DISP_G, C), jnp.float32),
                       pltpu.SemaphoreType.DMA],
    )
    return f(x2, tok)


def _gmm_body(te_ref, x_ref, ws_ref, w_ref, y_ref):
    xs = x_ref[...] * ws_ref[...]
    y_ref[...] = lax.dot_general(xs, w_ref[0], (((1,), (1,)), ((), ())),
                                 preferred_element_type=jnp.float32)


def _gmm(tile_e, xs, wsort, expert_w):
    grid_spec = pltpu.PrefetchScalarGridSpec(
        num_scalar_prefetch=1,
        grid=(NT,),
        in_specs=[pl.BlockSpec((TILE, C), lambda i, te: (i, 0)),
                  pl.BlockSpec((TILE, 1), lambda i, te: (i, 0)),
                  pl.BlockSpec((1, C, C), lambda i, te: (te[i], 0, 0))],
        out_specs=pl.BlockSpec((TILE, C), lambda i, te: (i, 0)),
    )
    return pl.pallas_call(
        _gmm_body,
        grid_spec=grid_spec,
        out_shape=jax.ShapeDtypeStruct((P, C), jnp.float32),
    )(tile_e, xs, wsort, expert_w)


def _combine_body(y_hbm, p0_hbm, p1_hbm, out_hbm, i0_v, i1_v, a_v, b_v, s0, s1):
    wid = lax.axis_index("s") * NC + lax.axis_index("c")
    base = wid * _CMB_B
    pltpu.sync_copy(p0_hbm.at[pl.ds(base, _CMB_B)], i0_v)
    pltpu.sync_copy(p1_hbm.at[pl.ds(base, _CMB_B)], i1_v)
    for g in range(_CMB_B // _CMB_G):
        ca = pltpu.async_copy(
            y_hbm.at[i0_v.at[pl.ds(g * _CMB_G, _CMB_G)]], a_v, s0)
        cb = pltpu.async_copy(
            y_hbm.at[i1_v.at[pl.ds(g * _CMB_G, _CMB_G)]], b_v, s1)
        ca.wait()
        cb.wait()
        for r in range(_CMB_G):
            def _add_row(c, carry, r=r):
                sl = pl.ds(c * 16, 16)
                a_v[r, sl] = a_v[r, sl] + b_v[r, sl]
                return carry
            lax.fori_loop(0, C // 16, _add_row, 0)
        pltpu.sync_copy(a_v, out_hbm.at[pl.ds(base + g * _CMB_G, _CMB_G)])


def _combine(y, pos0, pos1):
    f = pl.kernel(
        _combine_body,
        out_type=jax.ShapeDtypeStruct((NTOK, C), jnp.float32),
        mesh=plsc.VectorSubcoreMesh(core_axis_name="c", subcore_axis_name="s",
                                    num_cores=NC, num_subcores=NS),
        scratch_types=[pltpu.VMEM((_CMB_B,), jnp.int32),
                       pltpu.VMEM((_CMB_B,), jnp.int32),
                       pltpu.VMEM((_CMB_G, C), jnp.float32),
                       pltpu.VMEM((_CMB_G, C), jnp.float32),
                       pltpu.SemaphoreType.DMA,
                       pltpu.SemaphoreType.DMA],
    )
    return f(y, pos0, pos1)


def kernel(inputs, gate_w, expert_w):
    B, T, Cc = inputs.shape
    x2 = inputs.reshape(B * T, Cc)
    sel, w = _gate(x2, gate_w)
    tok, wsort, tile_e, pos0, pos1 = _route(sel, w)
    xs = _dispatch(x2, tok)
    y = _gmm(tile_e, xs, wsort, expert_w)
    out = _combine(y, pos0, pos1)
    return out.reshape(B, T, Cc)


# speedup vs baseline: 1.0924x; 1.0924x over previous
import functools

import jax
import jax.numpy as jnp
from jax import lax
from jax.experimental import pallas as pl
from jax.experimental.pallas import tpu as pltpu
from jax.experimental.pallas import tpu_sc as plsc

E = 8
K = 2
C = 2048
NTOK = 8192
NA = NTOK * K
TILE = 256
P = NA + E * TILE
NT = P // TILE
NC, NS = 2, 16
NW = NC * NS

GATE_TM = 1024

_DISP_B = P // NW
_DISP_G = 24
_CMB_B = NTOK // NW
_CMB_G = 8


def _gate_body(x_ref, gw_ref, sel_ref, w_ref):
    x = x_ref[...]
    logits = lax.dot_general(x, gw_ref[...], (((1,), (1,)), ((), ())),
                             preferred_element_type=jnp.float32)
    eiota = lax.broadcasted_iota(jnp.int32, logits.shape, 1)
    m1 = jnp.max(logits, axis=1, keepdims=True)
    a1 = jnp.min(jnp.where(logits == m1, eiota, E), axis=1, keepdims=True)
    l2 = jnp.where(eiota == a1, -jnp.inf, logits)
    m2 = jnp.max(l2, axis=1, keepdims=True)
    a2 = jnp.min(jnp.where(l2 == m2, eiota, E), axis=1, keepdims=True)
    z = jnp.exp(m2 - m1)
    sel_ref[...] = jnp.concatenate([a1, a2], axis=1)
    w_ref[...] = jnp.concatenate([1.0 / (1.0 + z), z / (1.0 + z)], axis=1)


def _gate(x2, gate_w):
    return pl.pallas_call(
        _gate_body,
        grid=(NTOK // GATE_TM,),
        in_specs=[pl.BlockSpec((GATE_TM, C), lambda i: (i, 0)),
                  pl.BlockSpec((E, C), lambda i: (0, 0))],
        out_specs=[pl.BlockSpec((GATE_TM, K), lambda i: (i, 0)),
                   pl.BlockSpec((GATE_TM, K), lambda i: (i, 0))],
        out_shape=[jax.ShapeDtypeStruct((NTOK, K), jnp.int32),
                   jax.ShapeDtypeStruct((NTOK, K), jnp.float32)],
    )(x2, gate_w)


def _route(sel, w):
    flat_e = sel.reshape(-1)
    onehot = (flat_e[:, None] == jnp.arange(E, dtype=jnp.int32)[None, :])
    cum = jnp.cumsum(onehot.astype(jnp.int32), axis=0)
    counts = cum[-1]
    rank = jnp.take_along_axis(cum, flat_e[:, None], axis=1)[:, 0] - 1
    aligned = ((counts + TILE - 1) // TILE) * TILE
    ends = jnp.cumsum(aligned)
    off = ends - aligned
    pos = off[flat_e] + rank
    tok = jnp.zeros((P,), jnp.int32).at[pos].set(
        jnp.arange(NA, dtype=jnp.int32) // K)
    wsort = jnp.zeros((P,), jnp.float32).at[pos].set(w.reshape(-1))
    tile_starts = jnp.arange(NT, dtype=jnp.int32) * TILE
    tile_e = jnp.minimum(
        jnp.searchsorted(ends, tile_starts, side="right"), E - 1
    ).astype(jnp.int32)
    pos2 = pos.reshape(NTOK, K).astype(jnp.int32)
    return tok, wsort.reshape(P, 1), tile_e, pos2[:, 0], pos2[:, 1]


def _dispatch_body(x_hbm, tok_hbm, xs_hbm, idx_v, rows_v0, rows_v1,
                   g0, g1, w0, w1):
    wid = lax.axis_index("s") * NC + lax.axis_index("c")
    base = wid * _DISP_B
    pltpu.sync_copy(tok_hbm.at[pl.ds(base, _DISP_B)], idx_v)
    bufs = [rows_v0, rows_v1]
    gsems = [g0, g1]
    wsems = [w0, w1]
    n = _DISP_B // _DISP_G

    def gather(g, b):
        return pltpu.async_copy(
            x_hbm.at[idx_v.at[pl.ds(g * _DISP_G, _DISP_G)]], bufs[b], gsems[b])

    def write(g, b):
        return pltpu.async_copy(
            bufs[b], xs_hbm.at[pl.ds(base + g * _DISP_G, _DISP_G)], wsems[b])

    gh = {0: gather(0, 0)}
    wh = {}
    for g in range(n):
        b = g & 1
        gh[g].wait()
        if g + 1 < n:
            if g - 1 >= 0:
                wh[g - 1].wait()
            gh[g + 1] = gather(g + 1, 1 - b)
        wh[g] = write(g, b)
    if n > 1:
        wh[n - 2].wait()
    wh[n - 1].wait()


def _dispatch(x2, tok):
    f = pl.kernel(
        _dispatch_body,
        out_type=jax.ShapeDtypeStruct((P, C), jnp.float32),
        mesh=plsc.VectorSubcoreMesh(core_axis_name="c", subcore_axis_name="s",
                                    num_cores=NC, num_subcores=NS),
        scratch_types=[pltpu.VMEM((_DISP_B,), jnp.int32),
                       pltpu.VMEM((_DISP_G, C), jnp.float32),
                       pltpu.VMEM((_DISP_G, C), jnp.float32),
                       pltpu.SemaphoreType.DMA,
                       pltpu.SemaphoreType.DMA,
                       pltpu.SemaphoreType.DMA,
                       pltpu.SemaphoreType.DMA],
    )
    return f(x2, tok)


def _gmm_body(te_ref, x_ref, ws_ref, w_ref, y_ref):
    xs = x_ref[...] * ws_ref[...]
    y_ref[...] = lax.dot_general(xs, w_ref[0], (((1,), (1,)), ((), ())),
                                 preferred_element_type=jnp.float32)


def _gmm(tile_e, xs, wsort, expert_w):
    grid_spec = pltpu.PrefetchScalarGridSpec(
        num_scalar_prefetch=1,
        grid=(NT,),
        in_specs=[pl.BlockSpec((TILE, C), lambda i, te: (i, 0)),
                  pl.BlockSpec((TILE, 1), lambda i, te: (i, 0)),
                  pl.BlockSpec((1, C, C), lambda i, te: (te[i], 0, 0))],
        out_specs=pl.BlockSpec((TILE, C), lambda i, te: (i, 0)),
    )
    return pl.pallas_call(
        _gmm_body,
        grid_spec=grid_spec,
        out_shape=jax.ShapeDtypeStruct((P, C), jnp.float32),
    )(tile_e, xs, wsort, expert_w)


def _combine_body(y_hbm, p0_hbm, p1_hbm, out_hbm, i0_v, i1_v,
                  a0_v, a1_v, b0_v, b1_v, ga0, ga1, gb0, gb1, w0, w1):
    wid = lax.axis_index("s") * NC + lax.axis_index("c")
    base = wid * _CMB_B
    pltpu.sync_copy(p0_hbm.at[pl.ds(base, _CMB_B)], i0_v)
    pltpu.sync_copy(p1_hbm.at[pl.ds(base, _CMB_B)], i1_v)
    abufs = [a0_v, a1_v]
    bbufs = [b0_v, b1_v]
    gasems = [ga0, ga1]
    gbsems = [gb0, gb1]
    wsems = [w0, w1]
    n = _CMB_B // _CMB_G

    def gathers(g, b):
        sl = pl.ds(g * _CMB_G, _CMB_G)
        return (pltpu.async_copy(y_hbm.at[i0_v.at[sl]], abufs[b], gasems[b]),
                pltpu.async_copy(y_hbm.at[i1_v.at[sl]], bbufs[b], gbsems[b]))

    gh = {0: gathers(0, 0)}
    wh = {}
    for g in range(n):
        b = g & 1
        gh[g][0].wait()
        gh[g][1].wait()
        if g + 1 < n:
            if g - 1 >= 0:
                wh[g - 1].wait()
            gh[g + 1] = gathers(g + 1, 1 - b)

        def _add(i, carry, a_v=abufs[b], b_v=bbufs[b]):
            r = lax.shift_right_logical(i, 7)
            sl = pl.ds((i & 127) * 16, 16)
            a_v[r, sl] = a_v[r, sl] + b_v[r, sl]
            return carry

        lax.fori_loop(0, _CMB_G * (C // 16), _add, 0)
        wh[g] = pltpu.async_copy(
            abufs[b], out_hbm.at[pl.ds(base + g * _CMB_G, _CMB_G)], wsems[b])
    if n > 1:
        wh[n - 2].wait()
    wh[n - 1].wait()


def _combine(y, pos0, pos1):
    f = pl.kernel(
        _combine_body,
        out_type=jax.ShapeDtypeStruct((NTOK, C), jnp.float32),
        mesh=plsc.VectorSubcoreMesh(core_axis_name="c", subcore_axis_name="s",
                                    num_cores=NC, num_subcores=NS),
        scratch_types=[pltpu.VMEM((_CMB_B,), jnp.int32),
                       pltpu.VMEM((_CMB_B,), jnp.int32),
                       pltpu.VMEM((_CMB_G, C), jnp.float32),
                       pltpu.VMEM((_CMB_G, C), jnp.float32),
                       pltpu.VMEM((_CMB_G, C), jnp.float32),
                       pltpu.VMEM((_CMB_G, C), jnp.float32),
                       pltpu.SemaphoreType.DMA,
                       pltpu.SemaphoreType.DMA,
                       pltpu.SemaphoreType.DMA,
                       pltpu.SemaphoreType.DMA,
                       pltpu.SemaphoreType.DMA,
                       pltpu.SemaphoreType.DMA],
    )
    return f(y, pos0, pos1)


def kernel(inputs, gate_w, expert_w):
    B, T, Cc = inputs.shape
    x2 = inputs.reshape(B * T, Cc)
    sel, w = _gate(x2, gate_w)
    tok, wsort, tile_e, pos0, pos1 = _route(sel, w)
    xs = _dispatch(x2, tok)
    y = _gmm(tile_e, xs, wsort, expert_w)
    out = _combine(y, pos0, pos1)
    return out.reshape(B, T, Cc)


# speedup vs baseline: 1.4129x; 1.2934x over previous
import functools

import jax
import jax.numpy as jnp
from jax import lax
from jax.experimental import pallas as pl
from jax.experimental.pallas import tpu as pltpu
from jax.experimental.pallas import tpu_sc as plsc

E = 8
K = 2
C = 2048
CH = C // 2
NTOK = 8192
TILE = 256
NC, NS = 2, 16
NW = NC * NS

NH = 2
NTOK_H = NTOK // NH
NA_H = NTOK_H * K
P_H = NA_H + E * TILE
NT_H = P_H // TILE

GATE_TM = 1024

_DISP_B = NTOK_H // NW
_DISP_G = 32
_DISP_NBUF = 3
_CMB_B = NTOK_H // NW
_CMB_G = 8
_CMB_NBUF = 3


def _pack_bf16(a, b):
    au = lax.bitcast_convert_type(a, jnp.uint16).astype(jnp.uint32)
    bu = lax.bitcast_convert_type(b, jnp.uint16).astype(jnp.uint32)
    return lax.bitcast_convert_type(au | (bu << 16), jnp.int32)


def _unpack_bf16(p):
    u = lax.bitcast_convert_type(p, jnp.uint32)
    lo = lax.bitcast_convert_type((u & 0xFFFF).astype(jnp.uint16),
                                  jnp.bfloat16)
    hi = lax.bitcast_convert_type((u >> 16).astype(jnp.uint16), jnp.bfloat16)
    return jnp.concatenate([lo, hi], axis=-1)


def _gate_body(x_ref, gw_ref, sel_ref, w_ref, xpk_ref):
    x = x_ref[...]
    xb = x.astype(jnp.bfloat16)
    xpk_ref[...] = _pack_bf16(xb[:, :CH], xb[:, CH:])
    logits = lax.dot_general(x, gw_ref[...], (((1,), (1,)), ((), ())),
                             preferred_element_type=jnp.float32)
    eiota = lax.broadcasted_iota(jnp.int32, logits.shape, 1)
    m1 = jnp.max(logits, axis=1, keepdims=True)
    a1 = jnp.min(jnp.where(logits == m1, eiota, E), axis=1, keepdims=True)
    l2 = jnp.where(eiota == a1, -jnp.inf, logits)
    m2 = jnp.max(l2, axis=1, keepdims=True)
    a2 = jnp.min(jnp.where(l2 == m2, eiota, E), axis=1, keepdims=True)
    z = jnp.exp(m2 - m1)
    sel_ref[...] = jnp.concatenate([a1, a2], axis=1)
    w_ref[...] = jnp.concatenate([1.0 / (1.0 + z), z / (1.0 + z)], axis=1)


def _gate(x2, gate_w):
    return pl.pallas_call(
        _gate_body,
        grid=(NTOK // GATE_TM,),
        in_specs=[pl.BlockSpec((GATE_TM, C), lambda i: (i, 0)),
                  pl.BlockSpec((E, C), lambda i: (0, 0))],
        out_specs=[pl.BlockSpec((GATE_TM, K), lambda i: (i, 0)),
                   pl.BlockSpec((GATE_TM, K), lambda i: (i, 0)),
                   pl.BlockSpec((GATE_TM, CH), lambda i: (i, 0))],
        out_shape=[jax.ShapeDtypeStruct((NTOK, K), jnp.int32),
                   jax.ShapeDtypeStruct((NTOK, K), jnp.float32),
                   jax.ShapeDtypeStruct((NTOK, CH), jnp.int32)],
    )(x2, gate_w)


def _route(sel_h):
    flat_e = sel_h.reshape(-1)
    onehot = (flat_e[:, None] == jnp.arange(E, dtype=jnp.int32)[None, :])
    cum = jnp.cumsum(onehot.astype(jnp.int32), axis=0)
    counts = cum[-1]
    rank = jnp.take_along_axis(cum, flat_e[:, None], axis=1)[:, 0] - 1
    aligned = ((counts + TILE - 1) // TILE) * TILE
    ends = jnp.cumsum(aligned)
    off = ends - aligned
    pos = off[flat_e] + rank
    tile_starts = jnp.arange(NT_H, dtype=jnp.int32) * TILE
    tile_e = jnp.minimum(
        jnp.searchsorted(ends, tile_starts, side="right"), E - 1
    ).astype(jnp.int32)
    pos2 = pos.reshape(NTOK_H, K).astype(jnp.int32)
    nds = _DISP_B // _DISP_G
    s0 = pos2[:, 0].reshape(NW, nds, 1, _DISP_G)
    s1 = pos2[:, 1].reshape(NW, nds, 1, _DISP_G)
    sidx = jnp.concatenate([s0, s1], axis=2).reshape(NW, 2 * nds, _DISP_G)
    nch = _CMB_B // _CMB_G
    p0 = pos2[:, 0].reshape(NW, nch, _CMB_G)
    p1 = pos2[:, 1].reshape(NW, nch, _CMB_G)
    pidx = jnp.concatenate([p0, p1], axis=2).reshape(-1)
    return tile_e, sidx, pidx


def _dispatch_body(h, x_hbm, sidx_hbm, xs_hbm, idx_v, *scr):
    nb = _DISP_NBUF
    bufs = scr[:nb]
    gsems = scr[nb:2 * nb]
    wsems = scr[2 * nb:3 * nb]
    wid = lax.axis_index("s") * NC + lax.axis_index("c")
    base = h * NTOK_H + wid * _DISP_B
    pltpu.sync_copy(sidx_hbm.at[wid], idx_v)
    n = _DISP_B // _DISP_G

    def read(g, b):
        return pltpu.async_copy(
            x_hbm.at[pl.ds(base + g * _DISP_G, _DISP_G)], bufs[b], gsems[b])

    def scatter(g, b):
        return (pltpu.async_copy(bufs[b], xs_hbm.at[idx_v.at[2 * g]],
                                 wsems[b]),
                pltpu.async_copy(bufs[b], xs_hbm.at[idx_v.at[2 * g + 1]],
                                 wsems[b]))

    ahead = 2
    gh, wh = {}, {}
    for j in range(min(ahead, n)):
        gh[j] = read(j, j % nb)
    for g in range(n):
        b = g % nb
        gh[g].wait()
        j = g + ahead
        if j < n:
            if j - nb >= 0:
                for hd in wh[j - nb]:
                    hd.wait()
            gh[j] = read(j, j % nb)
        wh[g] = scatter(g, b)
    for g in range(max(0, n - nb), n):
        if g in wh:
            for hd in wh[g]:
                hd.wait()


def _dispatch(xpk, sidx, h):
    scr = [pltpu.VMEM((2 * (_DISP_B // _DISP_G), _DISP_G), jnp.int32)]
    scr += [pltpu.VMEM((_DISP_G, CH), jnp.int32)] * _DISP_NBUF
    scr += [pltpu.SemaphoreType.DMA] * (2 * _DISP_NBUF)
    f = pl.kernel(
        functools.partial(_dispatch_body, h),
        out_type=jax.ShapeDtypeStruct((P_H, CH), jnp.int32),
        mesh=plsc.VectorSubcoreMesh(core_axis_name="c", subcore_axis_name="s",
                                    num_cores=NC, num_subcores=NS),
        scratch_types=scr,
    )
    return f(xpk, sidx)


def _gmm_body(te_ref, x_ref, w_ref, y_ref):
    u = lax.bitcast_convert_type(x_ref[...], jnp.uint32)
    lo = lax.bitcast_convert_type((u & 0xFFFF).astype(jnp.uint16),
                                  jnp.bfloat16)
    hi = lax.bitcast_convert_type((u >> 16).astype(jnp.uint16), jnp.bfloat16)
    wb = w_ref[0]
    acc = lax.dot_general(lo, wb[:, :CH], (((1,), (1,)), ((), ())),
                          preferred_element_type=jnp.float32)
    acc += lax.dot_general(hi, wb[:, CH:], (((1,), (1,)), ((), ())),
                           preferred_element_type=jnp.float32)
    yb = acc.astype(jnp.bfloat16)
    y_ref[...] = _pack_bf16(yb[:, :CH], yb[:, CH:])


def _gmm(tile_e, xs, expert_wbf):
    grid_spec = pltpu.PrefetchScalarGridSpec(
        num_scalar_prefetch=1,
        grid=(NT_H,),
        in_specs=[pl.BlockSpec((TILE, CH), lambda i, te: (i, 0)),
                  pl.BlockSpec((1, C, C), lambda i, te: (te[i], 0, 0))],
        out_specs=pl.BlockSpec((TILE, CH), lambda i, te: (i, 0)),
    )
    return pl.pallas_call(
        _gmm_body,
        grid_spec=grid_spec,
        out_shape=jax.ShapeDtypeStruct((P_H, CH), jnp.int32),
    )(tile_e, xs, expert_wbf)


def _combine_body(y_hbm, pidx_hbm, u0_hbm, u1_hbm, idx_v, *scr):
    nb = _CMB_NBUF
    bufs = scr[:nb]
    gsems = scr[nb:2 * nb]
    wsems = scr[2 * nb:3 * nb]
    wid = lax.axis_index("s") * NC + lax.axis_index("c")
    base = wid * _CMB_B
    g2 = 2 * _CMB_G
    pltpu.sync_copy(pidx_hbm.at[pl.ds(wid * (2 * _CMB_B), 2 * _CMB_B)], idx_v)
    n = _CMB_B // _CMB_G

    def gather(g, b):
        return pltpu.async_copy(
            y_hbm.at[idx_v.at[pl.ds(g * g2, g2)]], bufs[b], gsems[b])

    def write(g, b):
        dst = pl.ds(base + g * _CMB_G, _CMB_G)
        return (pltpu.async_copy(bufs[b].at[pl.ds(0, _CMB_G)],
                                 u0_hbm.at[dst], wsems[b]),
                pltpu.async_copy(bufs[b].at[pl.ds(_CMB_G, _CMB_G)],
                                 u1_hbm.at[dst], wsems[b]))

    ahead = 2
    gh, wh = {}, {}
    for j in range(min(ahead, n)):
        gh[j] = gather(j, j % nb)
    for g in range(n):
        b = g % nb
        gh[g].wait()
        j = g + ahead
        if j < n:
            if j - nb >= 0:
                for hd in wh[j - nb]:
                    hd.wait()
            gh[j] = gather(j, j % nb)
        wh[g] = write(g, b)
    for g in range(max(0, n - nb), n):
        if g in wh:
            for hd in wh[g]:
                hd.wait()


def _combine(y, pidx):
    scr = [pltpu.VMEM((2 * _CMB_B,), jnp.int32)]
    scr += [pltpu.VMEM((2 * _CMB_G, CH), jnp.int32)] * _CMB_NBUF
    scr += [pltpu.SemaphoreType.DMA] * (2 * _CMB_NBUF)
    f = pl.kernel(
        _combine_body,
        out_type=[jax.ShapeDtypeStruct((NTOK_H, CH), jnp.int32),
                  jax.ShapeDtypeStruct((NTOK_H, CH), jnp.int32)],
        mesh=plsc.VectorSubcoreMesh(core_axis_name="c", subcore_axis_name="s",
                                    num_cores=NC, num_subcores=NS),
        scratch_types=scr,
    )
    return f(y, pidx)


def _unpack_body(u0_ref, u1_ref, w_ref, o_ref):
    a = _unpack_bf16(u0_ref[...]).astype(jnp.float32)
    b = _unpack_bf16(u1_ref[...]).astype(jnp.float32)
    w = w_ref[...]
    o_ref[...] = w[:, 0:1] * a + w[:, 1:2] * b


def _unpack(u0, u1, w_h):
    return pl.pallas_call(
        _unpack_body,
        grid=(NTOK_H // GATE_TM,),
        in_specs=[pl.BlockSpec((GATE_TM, CH), lambda i: (i, 0)),
                  pl.BlockSpec((GATE_TM, CH), lambda i: (i, 0)),
                  pl.BlockSpec((GATE_TM, K), lambda i: (i, 0))],
        out_specs=pl.BlockSpec((GATE_TM, C), lambda i: (i, 0)),
        out_shape=jax.ShapeDtypeStruct((NTOK_H, C), jnp.float32),
    )(u0, u1, w_h)


def kernel(inputs, gate_w, expert_w):
    B, T, Cc = inputs.shape
    x2 = inputs.reshape(B * T, Cc)
    sel, w, xpk = _gate(x2, gate_w)
    expert_wbf = expert_w.astype(jnp.bfloat16)
    outs = []
    for h in range(NH):
        sl = slice(h * NTOK_H, (h + 1) * NTOK_H)
        tile_e, sidx, pidx = _route(sel[sl])
        xs = _dispatch(xpk, sidx, h)
        y = _gmm(tile_e, xs, expert_wbf)
        u0, u1 = _combine(y, pidx)
        outs.append(_unpack(u0, u1, w[sl]))
    return jnp.stack(outs).reshape(B, T, Cc)


# speedup vs baseline: 1.7518x; 1.2399x over previous
import functools

import jax
import jax.numpy as jnp
from jax import lax
from jax.experimental import pallas as pl
from jax.experimental.pallas import tpu as pltpu
from jax.experimental.pallas import tpu_sc as plsc

E = 8
K = 2
C = 2048
NTOK = 8192
NA = NTOK * K
TILE = 256
P = NA + E * TILE
NT = P // TILE
NC, NS = 2, 16
NW = NC * NS

GATE_TM = 1024

_DISP_B = NTOK // NW
_DISP_G = 32
_DISP_NBUF = 3
_CMB_B = NTOK // NW
_CMB_G = 8
_CMB_NBUF = 3


CH = C // 2


def _pack_bf16(a, b):
    au = lax.bitcast_convert_type(a, jnp.uint16).astype(jnp.uint32)
    bu = lax.bitcast_convert_type(b, jnp.uint16).astype(jnp.uint32)
    return lax.bitcast_convert_type(au | (bu << 16), jnp.int32)


def _unpack_bf16(p):
    u = lax.bitcast_convert_type(p, jnp.uint32)
    lo = lax.bitcast_convert_type((u & 0xFFFF).astype(jnp.uint16),
                                  jnp.bfloat16)
    hi = lax.bitcast_convert_type((u >> 16).astype(jnp.uint16), jnp.bfloat16)
    return jnp.concatenate([lo, hi], axis=-1)


def _gate_body(x_ref, gw_ref, sel_ref, w_ref, xpk_ref):
    x = x_ref[...]
    xb = x.astype(jnp.bfloat16)
    xpk_ref[...] = _pack_bf16(xb[:, :CH], xb[:, CH:])
    logits = lax.dot_general(x, gw_ref[...], (((1,), (1,)), ((), ())),
                             preferred_element_type=jnp.float32)
    eiota = lax.broadcasted_iota(jnp.int32, logits.shape, 1)
    m1 = jnp.max(logits, axis=1, keepdims=True)
    a1 = jnp.min(jnp.where(logits == m1, eiota, E), axis=1, keepdims=True)
    l2 = jnp.where(eiota == a1, -jnp.inf, logits)
    m2 = jnp.max(l2, axis=1, keepdims=True)
    a2 = jnp.min(jnp.where(l2 == m2, eiota, E), axis=1, keepdims=True)
    z = jnp.exp(m2 - m1)
    sel_ref[...] = jnp.concatenate([a1, a2], axis=1)
    w_ref[...] = jnp.concatenate([1.0 / (1.0 + z), z / (1.0 + z)], axis=1)


def _gate(x2, gate_w):
    return pl.pallas_call(
        _gate_body,
        grid=(NTOK // GATE_TM,),
        in_specs=[pl.BlockSpec((GATE_TM, C), lambda i: (i, 0)),
                  pl.BlockSpec((E, C), lambda i: (0, 0))],
        out_specs=[pl.BlockSpec((GATE_TM, K), lambda i: (i, 0)),
                   pl.BlockSpec((GATE_TM, K), lambda i: (i, 0)),
                   pl.BlockSpec((GATE_TM, CH), lambda i: (i, 0))],
        out_shape=[jax.ShapeDtypeStruct((NTOK, K), jnp.int32),
                   jax.ShapeDtypeStruct((NTOK, K), jnp.float32),
                   jax.ShapeDtypeStruct((NTOK, CH), jnp.int32)],
    )(x2, gate_w)


def _route(sel):
    flat_e = sel.reshape(-1)
    onehot = (flat_e[:, None] == jnp.arange(E, dtype=jnp.int32)[None, :])
    cum = jnp.cumsum(onehot.astype(jnp.int32), axis=0)
    counts = cum[-1]
    rank = jnp.take_along_axis(cum, flat_e[:, None], axis=1)[:, 0] - 1
    aligned = ((counts + TILE - 1) // TILE) * TILE
    ends = jnp.cumsum(aligned)
    off = ends - aligned
    pos = off[flat_e] + rank
    tile_starts = jnp.arange(NT, dtype=jnp.int32) * TILE
    tile_e = jnp.minimum(
        jnp.sum((tile_starts[:, None] >= ends[None, :]).astype(jnp.int32),
                axis=1),
        E - 1,
    ).astype(jnp.int32)
    pos2 = pos.reshape(NTOK, K).astype(jnp.int32)
    nds = _DISP_B // _DISP_G
    s0 = pos2[:, 0].reshape(NW, nds, 1, _DISP_G)
    s1 = pos2[:, 1].reshape(NW, nds, 1, _DISP_G)
    sidx = jnp.concatenate([s0, s1], axis=2).reshape(NW, 2 * nds, _DISP_G)
    nch = _CMB_B // _CMB_G
    p0 = pos2[:, 0].reshape(NW, nch, _CMB_G)
    p1 = pos2[:, 1].reshape(NW, nch, _CMB_G)
    pidx = jnp.concatenate([p0, p1], axis=2).reshape(-1)
    return tile_e, sidx, pidx


def _dispatch_body(x_hbm, sidx_hbm, xs_hbm, idx_v, *scr):
    nb = _DISP_NBUF
    bufs = scr[:nb]
    gsems = scr[nb:2 * nb]
    wsems = scr[2 * nb:3 * nb]
    wid = lax.axis_index("s") * NC + lax.axis_index("c")
    base = wid * _DISP_B
    pltpu.sync_copy(sidx_hbm.at[wid], idx_v)
    n = _DISP_B // _DISP_G

    def read(g, b):
        return pltpu.async_copy(
            x_hbm.at[pl.ds(base + g * _DISP_G, _DISP_G)], bufs[b], gsems[b])

    def scatter(g, b):
        return (pltpu.async_copy(bufs[b], xs_hbm.at[idx_v.at[2 * g]],
                                 wsems[b]),
                pltpu.async_copy(bufs[b], xs_hbm.at[idx_v.at[2 * g + 1]],
                                 wsems[b]))

    ahead = 2
    gh, wh = {}, {}
    for j in range(min(ahead, n)):
        gh[j] = read(j, j % nb)
    for g in range(n):
        b = g % nb
        gh[g].wait()
        j = g + ahead
        if j < n:
            if j - nb >= 0:
                for h in wh[j - nb]:
                    h.wait()
            gh[j] = read(j, j % nb)
        wh[g] = scatter(g, b)
    for g in range(max(0, n - nb), n):
        if g in wh:
            for h in wh[g]:
                h.wait()


def _dispatch(x2, sidx):
    scr = [pltpu.VMEM((2 * (_DISP_B // _DISP_G), _DISP_G), jnp.int32)]
    scr += [pltpu.VMEM((_DISP_G, CH), jnp.int32)] * _DISP_NBUF
    scr += [pltpu.SemaphoreType.DMA] * (2 * _DISP_NBUF)
    f = pl.kernel(
        _dispatch_body,
        out_type=jax.ShapeDtypeStruct((P, CH), jnp.int32),
        mesh=plsc.VectorSubcoreMesh(core_axis_name="c", subcore_axis_name="s",
                                    num_cores=NC, num_subcores=NS),
        scratch_types=scr,
    )
    return f(x2, sidx)


def _gmm_body(te_ref, x_ref, w_ref, y_ref):
    u = lax.bitcast_convert_type(x_ref[...], jnp.uint32)
    lo = lax.bitcast_convert_type((u & 0xFFFF).astype(jnp.uint16),
                                  jnp.bfloat16)
    hi = lax.bitcast_convert_type((u >> 16).astype(jnp.uint16), jnp.bfloat16)
    wb = w_ref[0]
    acc = lax.dot_general(lo, wb[:, :CH], (((1,), (1,)), ((), ())),
                          preferred_element_type=jnp.float32)
    acc += lax.dot_general(hi, wb[:, CH:], (((1,), (1,)), ((), ())),
                           preferred_element_type=jnp.float32)
    yb = acc.astype(jnp.bfloat16)
    y_ref[...] = _pack_bf16(yb[:, :CH], yb[:, CH:])


def _gmm(tile_e, xs, expert_w):
    grid_spec = pltpu.PrefetchScalarGridSpec(
        num_scalar_prefetch=1,
        grid=(NT,),
        in_specs=[pl.BlockSpec((TILE, CH), lambda i, te: (i, 0)),
                  pl.BlockSpec((1, C, C), lambda i, te: (te[i], 0, 0))],
        out_specs=pl.BlockSpec((TILE, CH), lambda i, te: (i, 0)),
    )
    return pl.pallas_call(
        _gmm_body,
        grid_spec=grid_spec,
        out_shape=jax.ShapeDtypeStruct((P, CH), jnp.int32),
    )(tile_e, xs, expert_w)


def _combine_body(y_hbm, pidx_hbm, u0_hbm, u1_hbm, idx_v, *scr):
    nb = _CMB_NBUF
    bufs = scr[:nb]
    gsems = scr[nb:2 * nb]
    wsems = scr[2 * nb:3 * nb]
    wid = lax.axis_index("s") * NC + lax.axis_index("c")
    base = wid * _CMB_B
    g2 = 2 * _CMB_G
    pltpu.sync_copy(pidx_hbm.at[pl.ds(wid * (2 * _CMB_B), 2 * _CMB_B)], idx_v)
    n = _CMB_B // _CMB_G

    def gather(g, b):
        return pltpu.async_copy(
            y_hbm.at[idx_v.at[pl.ds(g * g2, g2)]], bufs[b], gsems[b])

    def write(g, b):
        dst = pl.ds(base + g * _CMB_G, _CMB_G)
        return (pltpu.async_copy(bufs[b].at[pl.ds(0, _CMB_G)],
                                 u0_hbm.at[dst], wsems[b]),
                pltpu.async_copy(bufs[b].at[pl.ds(_CMB_G, _CMB_G)],
                                 u1_hbm.at[dst], wsems[b]))

    ahead = 2
    gh, wh = {}, {}
    for j in range(min(ahead, n)):
        gh[j] = gather(j, j % nb)
    for g in range(n):
        b = g % nb
        gh[g].wait()
        j = g + ahead
        if j < n:
            if j - nb >= 0:
                for h in wh[j - nb]:
                    h.wait()
            gh[j] = gather(j, j % nb)
        wh[g] = write(g, b)
    for g in range(max(0, n - nb), n):
        if g in wh:
            for h in wh[g]:
                h.wait()


def _combine(y, pidx):
    scr = [pltpu.VMEM((2 * _CMB_B,), jnp.int32)]
    scr += [pltpu.VMEM((2 * _CMB_G, CH), jnp.int32)] * _CMB_NBUF
    scr += [pltpu.SemaphoreType.DMA] * (2 * _CMB_NBUF)
    f = pl.kernel(
        _combine_body,
        out_type=[jax.ShapeDtypeStruct((NTOK, CH), jnp.int32),
                  jax.ShapeDtypeStruct((NTOK, CH), jnp.int32)],
        mesh=plsc.VectorSubcoreMesh(core_axis_name="c", subcore_axis_name="s",
                                    num_cores=NC, num_subcores=NS),
        scratch_types=scr,
    )
    return f(y, pidx)


def _unpack_body(u0_ref, u1_ref, w_ref, o_ref):
    a = _unpack_bf16(u0_ref[...]).astype(jnp.float32)
    b = _unpack_bf16(u1_ref[...]).astype(jnp.float32)
    w = w_ref[...]
    o_ref[...] = w[:, 0:1] * a + w[:, 1:2] * b


def _unpack(u0, u1, w):
    return pl.pallas_call(
        _unpack_body,
        grid=(NTOK // GATE_TM,),
        in_specs=[pl.BlockSpec((GATE_TM, CH), lambda i: (i, 0)),
                  pl.BlockSpec((GATE_TM, CH), lambda i: (i, 0)),
                  pl.BlockSpec((GATE_TM, K), lambda i: (i, 0))],
        out_specs=pl.BlockSpec((GATE_TM, C), lambda i: (i, 0)),
        out_shape=jax.ShapeDtypeStruct((NTOK, C), jnp.float32),
    )(u0, u1, w)


def kernel(inputs, gate_w, expert_w):
    B, T, Cc = inputs.shape
    x2 = inputs.reshape(B * T, Cc)
    sel, w, xpk = _gate(x2, gate_w)
    tile_e, sidx, pidx = _route(sel)
    xs = _dispatch(xpk, sidx)
    y = _gmm(tile_e, xs, expert_w.astype(jnp.bfloat16))
    u0, u1 = _combine(y, pidx)
    return _unpack(u0, u1, w).reshape(B, T, Cc)
